# Initial kernel scaffold; baseline (speedup 1.0000x reference)
import jax
import jax.numpy as jnp
from jax.experimental import pallas as pl


def kernel(x, max_pool_indices, up_neigh_indices, down_indices, W, b):
    raise NotImplementedError("write your pallas kernel here")


# speedup vs baseline: 557.9897x; 557.9897x over previous
import functools

import jax
import jax.numpy as jnp
from jax import lax
from jax.experimental import pallas as pl
from jax.experimental.pallas import tpu as pltpu
from jax.experimental.pallas import tpu_sc as plsc

N_VERT = 163842
N_RAW = 40962
NEIGH = 7
IN_F = 128
OUT_F = 64
BATCH = 4

NC = 2
NS = 16
NW = NC * NS

CH = 2048
NCHUNK = 21
W_P = CH * NCHUNK

P1 = W_P // NW

HALF0 = 81920
HALF1 = N_VERT - HALF0
VBUF = 81936

ROWS = BATCH * OUT_F
TASKS = 2 * OUT_F // NW


def _neigh_gather_kernel(up_hbm, down_hbm, nt_hbm, dbuf, idx2, val2, sem):
    wid = lax.axis_index("s") * NC + lax.axis_index("c")
    base = wid * P1
    pltpu.sync_copy(down_hbm.at[pl.ds(base, P1)], dbuf.at[pl.ds(0, P1)])
    zero16 = jnp.zeros((16,), jnp.int32)
    for k in range(12):
        dbuf[pl.ds(P1 + k * 16, 16)] = zero16

    for j in range(NEIGH):
        def fill(i, carry, j=j):
            d = dbuf[pl.ds(i * 16, 16)]
            idx2[i // 8, pl.ds((i % 8) * 16, 16)] = d * NEIGH + j
            return carry
        lax.fori_loop(0, 96, fill, 0, unroll=4)
        for r in range(11):
            pltpu.async_copy(up_hbm.at[idx2.at[r]], val2.at[r], sem).wait()
        for r in range(10):
            pltpu.sync_copy(val2.at[r], nt_hbm.at[j, pl.ds(base + r * 128, 128)])
        pltpu.sync_copy(val2.at[10, pl.ds(0, 64)],
                        nt_hbm.at[j, pl.ds(base + 1280, 64)])
    for r in range(10):
        pltpu.sync_copy(val2.at[r], nt_hbm.at[7, pl.ds(base + r * 128, 128)])
    pltpu.sync_copy(val2.at[10, pl.ds(0, 64)],
                    nt_hbm.at[7, pl.ds(base + 1280, 64)])


def _neigh_gather(up_flat, down_p):
    k = functools.partial(
        pl.kernel,
        mesh=plsc.VectorSubcoreMesh(core_axis_name="c", subcore_axis_name="s"),
        out_type=jax.ShapeDtypeStruct((8, W_P), jnp.int32),
        scratch_types=[
            pltpu.VMEM((1536,), jnp.int32),
            pltpu.VMEM((12, 128), jnp.int32),
            pltpu.VMEM((12, 128), jnp.int32),
            pltpu.SemaphoreType.DMA,
        ],
        compiler_params=pltpu.CompilerParams(use_tc_tiling_on_sc=False,
                                             needs_layout_passes=False),
    )(_neigh_gather_kernel)
    return k(up_flat, down_p)


def _fc_vi_kernel(x_ref, w_ref, b_ref, mpi_ref, nt_ref, xfc_ref, vi_ref):
    xb = x_ref[0]
    acc = jnp.dot(w_ref[...], xb, preferred_element_type=jnp.float32)
    acc = acc + b_ref[...]
    mpib = mpi_ref[0]
    nt = nt_ref[...]
    vi = jnp.broadcast_to(nt[0:1], (OUT_F, CH))
    for j in range(1, NEIGH):
        vi = jnp.where(mpib == j, nt[j:j + 1], vi)
    col = pl.program_id(1) * CH + lax.broadcasted_iota(jnp.int32, (OUT_F, CH), 1)
    vi = jnp.where(col < N_RAW, vi, -1)
    xfc_ref[0] = acc
    vi_ref[0] = vi


def _fc_vi(x, w, b2, mpi, neigh_t):
    grid = (BATCH, NCHUNK)
    return pl.pallas_call(
        _fc_vi_kernel,
        grid=grid,
        in_specs=[
            pl.BlockSpec((1, IN_F, CH), lambda s, c: (s, 0, c)),
            pl.BlockSpec((OUT_F, IN_F), lambda s, c: (0, 0)),
            pl.BlockSpec((OUT_F, 1), lambda s, c: (0, 0)),
            pl.BlockSpec((1, OUT_F, CH), lambda s, c: (s, 0, c)),
            pl.BlockSpec((8, CH), lambda s, c: (0, c)),
        ],
        out_specs=[
            pl.BlockSpec((1, OUT_F, CH), lambda s, c: (s, 0, c)),
            pl.BlockSpec((1, OUT_F, CH), lambda s, c: (s, 0, c)),
        ],
        out_shape=[
            jax.ShapeDtypeStruct((BATCH, OUT_F, W_P), jnp.float32),
            jax.ShapeDtypeStruct((BATCH, OUT_F, W_P), jnp.int32),
        ],
    )(x, w, b2, mpi, neigh_t)


def _scatter_kernel(vi_hbm, xf_hbm, y_hbm, vbuf, vich, xfch, s0, s1, s2, s3, so):
    wid = lax.axis_index("s") * NC + lax.axis_index("c")
    sems = ((s0, s1), (s2, s3))
    zero16 = jnp.zeros((16,), jnp.float32)

    def issue(row, ci, slot):
        pltpu.async_copy(vi_hbm.at[row, pl.ds(ci * CH, CH)],
                         vich.at[slot], sems[0][slot])
        pltpu.async_copy(xf_hbm.at[row, pl.ds(ci * CH, CH)],
                         xfch.at[slot], sems[1][slot])

    def wait(slot):
        pltpu.make_async_copy(vi_hbm.at[0, pl.ds(0, CH)],
                              vich.at[slot], sems[0][slot]).wait()
        pltpu.make_async_copy(xf_hbm.at[0, pl.ds(0, CH)],
                              xfch.at[slot], sems[1][slot]).wait()

    for t in range(TASKS):
        half = t & 1
        f = wid * (TASKS // 2) + (t >> 1)
        lo = HALF0 * half
        ln = HALF1 if half else HALF0

        def zfill(i, carry):
            base = i * 64
            vbuf[pl.ds(base, 16)] = zero16
            vbuf[pl.ds(base + 16, 16)] = zero16
            vbuf[pl.ds(base + 32, 16)] = zero16
            vbuf[pl.ds(base + 48, 16)] = zero16
            return carry
        lax.fori_loop(0, VBUF // 64, zfill, 0)

        def process(slot, ci):
            def inner(k, carry):
                o = k * 32
                viv = vich[slot, pl.ds(o, 16)]
                xfv = xfch[slot, pl.ds(o, 16)]
                rel = viv - lo
                m = (rel >= 0) & (rel < ln)
                plsc.store_scatter(vbuf, [rel], xfv, mask=m)
                viv2 = vich[slot, pl.ds(o + 16, 16)]
                xfv2 = xfch[slot, pl.ds(o + 16, 16)]
                rel2 = viv2 - lo
                m2 = (rel2 >= 0) & (rel2 < ln)
                plsc.store_scatter(vbuf, [rel2], xfv2, mask=m2)
                return carry
            lax.fori_loop(0, CH // 32, inner, 0)

        for s in range(BATCH):
            row = s * OUT_F + f
            issue(row, 0, 0)

            def outer(ci2, carry, row=row):
                for bslot in range(2):
                    ci = ci2 + bslot
                    issue(row, ci + 1, 1 - bslot)
                    wait(bslot)
                    process(bslot, ci)
                return carry
            lax.fori_loop(0, (NCHUNK - 1) // 2,
                          lambda i, c, fn=outer: fn(i * 2, c), 0)

            wait(0)
            process(0, NCHUNK - 1)

        for s in range(BATCH):
            pltpu.async_copy(vbuf.at[pl.ds(0, ln)],
                             y_hbm.at[s * OUT_F + f, pl.ds(lo, ln)], so)
        for s in range(BATCH):
            pltpu.make_async_copy(vbuf.at[pl.ds(0, ln)],
                                  y_hbm.at[s * OUT_F + f, pl.ds(lo, ln)],
                                  so).wait()


def _scatter(vi2, xf2):
    k = functools.partial(
        pl.kernel,
        mesh=plsc.VectorSubcoreMesh(core_axis_name="c", subcore_axis_name="s"),
        out_type=jax.ShapeDtypeStruct((ROWS, N_VERT), jnp.float32),
        scratch_types=[
            pltpu.VMEM((VBUF,), jnp.float32),
            pltpu.VMEM((2, CH), jnp.int32),
            pltpu.VMEM((2, CH), jnp.float32),
            pltpu.SemaphoreType.DMA,
            pltpu.SemaphoreType.DMA,
            pltpu.SemaphoreType.DMA,
            pltpu.SemaphoreType.DMA,
            pltpu.SemaphoreType.DMA,
        ],
        compiler_params=pltpu.CompilerParams(use_tc_tiling_on_sc=False,
                                             needs_layout_passes=False),
    )(_scatter_kernel)
    return k(vi2, xf2)


@jax.jit
def kernel(x, max_pool_indices, up_neigh_indices, down_indices, W, b):
    up_flat = jnp.reshape(up_neigh_indices.astype(jnp.int32), (-1,))
    down_p = jnp.concatenate(
        [down_indices.astype(jnp.int32),
         jnp.zeros((W_P - N_RAW,), jnp.int32)])
    neigh_t = _neigh_gather(up_flat, down_p)

    xfc_p, vi_p = _fc_vi(x, W, b.reshape(OUT_F, 1),
                         max_pool_indices.astype(jnp.int32), neigh_t)

    vi2 = jnp.reshape(vi_p, (ROWS, W_P))
    xf2 = jnp.reshape(xfc_p, (ROWS, W_P))
    y = _scatter(vi2, xf2)
    return jnp.reshape(y, (BATCH, OUT_F, N_VERT))


# speedup vs baseline: 575.9236x; 1.0321x over previous
import functools

import jax
import jax.numpy as jnp
from jax import lax
from jax.experimental import pallas as pl
from jax.experimental.pallas import tpu as pltpu
from jax.experimental.pallas import tpu_sc as plsc

N_VERT = 163842
N_RAW = 40962
NEIGH = 7
IN_F = 128
OUT_F = 64
BATCH = 4

NC = 2
NS = 16
NW = NC * NS

CH = 2048
NCHUNK = 21
W_P = CH * NCHUNK

P1 = W_P // NW

HALF0 = 81920
HALF1 = N_VERT - HALF0
VBUF = 81936

ROWS = BATCH * OUT_F
TASKS = 2 * OUT_F // NW


def _neigh_gather_kernel(up_hbm, down_hbm, nt_hbm, dbuf, idx2, val2, sem, sem2):
    wid = lax.axis_index("s") * NC + lax.axis_index("c")
    base = wid * P1
    pltpu.sync_copy(down_hbm.at[pl.ds(base, P1)], dbuf.at[pl.ds(0, P1)])
    zero16 = jnp.zeros((16,), jnp.int32)
    for k in range(12):
        dbuf[pl.ds(P1 + k * 16, 16)] = zero16

    for j in range(NEIGH):
        def fill(i, carry, j=j):
            d = dbuf[pl.ds(i * 16, 16)]
            idx2[i // 8, pl.ds((i % 8) * 16, 16)] = d * NEIGH + j
            return carry
        lax.fori_loop(0, 96, fill, 0, unroll=4)
        for r in range(11):
            pltpu.async_copy(up_hbm.at[idx2.at[r]], val2.at[r], sem)
        for r in range(11):
            pltpu.make_async_copy(up_hbm.at[idx2.at[r]], val2.at[r], sem).wait()
        for r in range(10):
            pltpu.async_copy(val2.at[r],
                             nt_hbm.at[j, pl.ds(base + r * 128, 128)], sem2)
        pltpu.async_copy(val2.at[10, pl.ds(0, 64)],
                         nt_hbm.at[j, pl.ds(base + 1280, 64)], sem2)
        if j == 6:
            for r in range(10):
                pltpu.async_copy(val2.at[r],
                                 nt_hbm.at[7, pl.ds(base + r * 128, 128)], sem2)
            pltpu.async_copy(val2.at[10, pl.ds(0, 64)],
                             nt_hbm.at[7, pl.ds(base + 1280, 64)], sem2)
        for r in range(10):
            pltpu.make_async_copy(
                val2.at[r], nt_hbm.at[j, pl.ds(base + r * 128, 128)],
                sem2).wait()
        pltpu.make_async_copy(val2.at[10, pl.ds(0, 64)],
                              nt_hbm.at[j, pl.ds(base + 1280, 64)], sem2).wait()
        if j == 6:
            for r in range(10):
                pltpu.make_async_copy(
                    val2.at[r], nt_hbm.at[7, pl.ds(base + r * 128, 128)],
                    sem2).wait()
            pltpu.make_async_copy(
                val2.at[10, pl.ds(0, 64)],
                nt_hbm.at[7, pl.ds(base + 1280, 64)], sem2).wait()


def _neigh_gather(up_flat, down_p):
    k = functools.partial(
        pl.kernel,
        mesh=plsc.VectorSubcoreMesh(core_axis_name="c", subcore_axis_name="s"),
        out_type=jax.ShapeDtypeStruct((8, W_P), jnp.int32),
        scratch_types=[
            pltpu.VMEM((1536,), jnp.int32),
            pltpu.VMEM((12, 128), jnp.int32),
            pltpu.VMEM((12, 128), jnp.int32),
            pltpu.SemaphoreType.DMA,
            pltpu.SemaphoreType.DMA,
        ],
        compiler_params=pltpu.CompilerParams(use_tc_tiling_on_sc=False,
                                             needs_layout_passes=False),
    )(_neigh_gather_kernel)
    return k(up_flat, down_p)


def _fc_vi_kernel(x_ref, w_ref, b_ref, mpi_ref, nt_ref, xfc_ref, vi_ref):
    xb = x_ref[0]
    acc = jnp.dot(w_ref[...], xb, preferred_element_type=jnp.float32)
    acc = acc + b_ref[...]
    mpib = mpi_ref[0]
    nt = nt_ref[...]
    vi = jnp.broadcast_to(nt[0:1], (OUT_F, CH))
    for j in range(1, NEIGH):
        vi = jnp.where(mpib == j, nt[j:j + 1], vi)
    col = pl.program_id(1) * CH + lax.broadcasted_iota(jnp.int32, (OUT_F, CH), 1)
    vi = jnp.where(col < N_RAW, vi, -1)
    xfc_ref[0] = acc
    vi_ref[0] = vi


def _fc_vi(x, w, b2, mpi, neigh_t):
    grid = (BATCH, NCHUNK)
    return pl.pallas_call(
        _fc_vi_kernel,
        grid=grid,
        in_specs=[
            pl.BlockSpec((1, IN_F, CH), lambda s, c: (s, 0, c)),
            pl.BlockSpec((OUT_F, IN_F), lambda s, c: (0, 0)),
            pl.BlockSpec((OUT_F, 1), lambda s, c: (0, 0)),
            pl.BlockSpec((1, OUT_F, CH), lambda s, c: (s, 0, c)),
            pl.BlockSpec((8, CH), lambda s, c: (0, c)),
        ],
        out_specs=[
            pl.BlockSpec((1, OUT_F, CH), lambda s, c: (s, 0, c)),
            pl.BlockSpec((1, OUT_F, CH), lambda s, c: (s, 0, c)),
        ],
        out_shape=[
            jax.ShapeDtypeStruct((BATCH, OUT_F, W_P), jnp.float32),
            jax.ShapeDtypeStruct((BATCH, OUT_F, W_P), jnp.int32),
        ],
    )(x, w, b2, mpi, neigh_t)


def _scatter_kernel(vi_hbm, xf_hbm, y_hbm, vbuf, vich, xfch, s0, s1, s2, s3, so):
    wid = lax.axis_index("s") * NC + lax.axis_index("c")
    sems = ((s0, s1), (s2, s3))
    zero16 = jnp.zeros((16,), jnp.float32)

    def issue(sample, f, ci, slot):
        pltpu.async_copy(vi_hbm.at[sample, f, pl.ds(ci * CH, CH)],
                         vich.at[slot], sems[0][slot])
        pltpu.async_copy(xf_hbm.at[sample, f, pl.ds(ci * CH, CH)],
                         xfch.at[slot], sems[1][slot])

    def wait(slot):
        pltpu.make_async_copy(vi_hbm.at[0, 0, pl.ds(0, CH)],
                              vich.at[slot], sems[0][slot]).wait()
        pltpu.make_async_copy(xf_hbm.at[0, 0, pl.ds(0, CH)],
                              xfch.at[slot], sems[1][slot]).wait()

    for t in range(TASKS):
        half = t & 1
        f = wid * (TASKS // 2) + (t >> 1)
        lo = HALF0 * half
        ln = HALF1 if half else HALF0

        def zfill(i, carry):
            base = i * 64
            vbuf[pl.ds(base, 16)] = zero16
            vbuf[pl.ds(base + 16, 16)] = zero16
            vbuf[pl.ds(base + 32, 16)] = zero16
            vbuf[pl.ds(base + 48, 16)] = zero16
            return carry
        lax.fori_loop(0, VBUF // 64, zfill, 0)

        def process(slot, ci):
            def body(k, carry):
                o = k * 64
                for u in range(4):
                    viv = vich[slot, pl.ds(o + u * 16, 16)]
                    xfv = xfch[slot, pl.ds(o + u * 16, 16)]
                    rel = viv - lo
                    m = rel.astype(jnp.uint32) < jnp.uint32(ln)
                    plsc.store_scatter(vbuf, [rel], xfv, mask=m)
                return carry
            lax.fori_loop(0, CH // 64, body, 0)

        for s in range(BATCH):
            issue(s, f, 0, 0)

            def outer(ci2, carry, s=s, f=f):
                for bslot in range(2):
                    ci = ci2 + bslot
                    issue(s, f, ci + 1, 1 - bslot)
                    wait(bslot)
                    process(bslot, ci)
                return carry
            lax.fori_loop(0, (NCHUNK - 1) // 2,
                          lambda i, c, fn=outer: fn(i * 2, c), 0)

            wait(0)
            process(0, NCHUNK - 1)

        for s in range(BATCH):
            pltpu.async_copy(vbuf.at[pl.ds(0, ln)],
                             y_hbm.at[s, f, pl.ds(lo, ln)], so)
        for s in range(BATCH):
            pltpu.make_async_copy(vbuf.at[pl.ds(0, ln)],
                                  y_hbm.at[s, f, pl.ds(lo, ln)], so).wait()


def _scatter(vi2, xf2):
    k = functools.partial(
        pl.kernel,
        mesh=plsc.VectorSubcoreMesh(core_axis_name="c", subcore_axis_name="s"),
        out_type=jax.ShapeDtypeStruct((BATCH, OUT_F, N_VERT), jnp.float32),
        scratch_types=[
            pltpu.VMEM((VBUF,), jnp.float32),
            pltpu.VMEM((2, CH), jnp.int32),
            pltpu.VMEM((2, CH), jnp.float32),
            pltpu.SemaphoreType.DMA,
            pltpu.SemaphoreType.DMA,
            pltpu.SemaphoreType.DMA,
            pltpu.SemaphoreType.DMA,
            pltpu.SemaphoreType.DMA,
        ],
        compiler_params=pltpu.CompilerParams(use_tc_tiling_on_sc=False,
                                             needs_layout_passes=False),
    )(_scatter_kernel)
    return k(vi2, xf2)


@jax.jit
def kernel(x, max_pool_indices, up_neigh_indices, down_indices, W, b):
    up_flat = jnp.reshape(up_neigh_indices.astype(jnp.int32), (-1,))
    down_p = jnp.concatenate(
        [down_indices.astype(jnp.int32),
         jnp.zeros((W_P - N_RAW,), jnp.int32)])
    neigh_t = _neigh_gather(up_flat, down_p)

    xfc_p, vi_p = _fc_vi(x, W, b.reshape(OUT_F, 1),
                         max_pool_indices.astype(jnp.int32), neigh_t)

    return _scatter(vi_p, xfc_p)


# speedup vs baseline: 576.0315x; 1.0002x over previous
import functools

import jax
import jax.numpy as jnp
from jax import lax
from jax.experimental import pallas as pl
from jax.experimental.pallas import tpu as pltpu
from jax.experimental.pallas import tpu_sc as plsc

N_VERT = 163842
N_RAW = 40962
NEIGH = 7
IN_F = 128
OUT_F = 64
BATCH = 4

NC = 2
NS = 16
NW = NC * NS

CH = 2048
NCHUNK = 21
W_P = CH * NCHUNK

P1 = W_P // NW

HALF0 = 81920
HALF1 = N_VERT - HALF0
VBUF = 81936

ROWS = BATCH * OUT_F
TASKS = 2 * OUT_F // NW


def _neigh_gather_kernel(up_hbm, down_hbm, nt_hbm, dbuf, idx2, val2, sem, sem2):
    wid = lax.axis_index("s") * NC + lax.axis_index("c")
    base = wid * P1
    pltpu.sync_copy(down_hbm.at[pl.ds(base, P1)], dbuf.at[pl.ds(0, P1)])
    zero16 = jnp.zeros((16,), jnp.int32)
    for k in range(12):
        dbuf[pl.ds(P1 + k * 16, 16)] = zero16

    for j in range(NEIGH):
        def fill(i, carry, j=j):
            d = dbuf[pl.ds(i * 16, 16)]
            idx2[i // 8, pl.ds((i % 8) * 16, 16)] = d * NEIGH + j
            return carry
        lax.fori_loop(0, 96, fill, 0, unroll=4)
        for r in range(11):
            pltpu.async_copy(up_hbm.at[idx2.at[r]], val2.at[r], sem)
        for r in range(11):
            pltpu.make_async_copy(up_hbm.at[idx2.at[r]], val2.at[r], sem).wait()
        for r in range(10):
            pltpu.async_copy(val2.at[r],
                             nt_hbm.at[j, pl.ds(base + r * 128, 128)], sem2)
        pltpu.async_copy(val2.at[10, pl.ds(0, 64)],
                         nt_hbm.at[j, pl.ds(base + 1280, 64)], sem2)
        if j == 6:
            for r in range(10):
                pltpu.async_copy(val2.at[r],
                                 nt_hbm.at[7, pl.ds(base + r * 128, 128)], sem2)
            pltpu.async_copy(val2.at[10, pl.ds(0, 64)],
                             nt_hbm.at[7, pl.ds(base + 1280, 64)], sem2)
        for r in range(10):
            pltpu.make_async_copy(
                val2.at[r], nt_hbm.at[j, pl.ds(base + r * 128, 128)],
                sem2).wait()
        pltpu.make_async_copy(val2.at[10, pl.ds(0, 64)],
                              nt_hbm.at[j, pl.ds(base + 1280, 64)], sem2).wait()
        if j == 6:
            for r in range(10):
                pltpu.make_async_copy(
                    val2.at[r], nt_hbm.at[7, pl.ds(base + r * 128, 128)],
                    sem2).wait()
            pltpu.make_async_copy(
                val2.at[10, pl.ds(0, 64)],
                nt_hbm.at[7, pl.ds(base + 1280, 64)], sem2).wait()


def _neigh_gather(up_flat, down_p):
    k = functools.partial(
        pl.kernel,
        mesh=plsc.VectorSubcoreMesh(core_axis_name="c", subcore_axis_name="s"),
        out_type=jax.ShapeDtypeStruct((8, W_P), jnp.int32),
        scratch_types=[
            pltpu.VMEM((1536,), jnp.int32),
            pltpu.VMEM((12, 128), jnp.int32),
            pltpu.VMEM((12, 128), jnp.int32),
            pltpu.SemaphoreType.DMA,
            pltpu.SemaphoreType.DMA,
        ],
        compiler_params=pltpu.CompilerParams(use_tc_tiling_on_sc=False,
                                             needs_layout_passes=False),
    )(_neigh_gather_kernel)
    return k(up_flat, down_p)


def _fc_vi_kernel(x_ref, w_ref, b_ref, mpi_ref, nt_ref, xfc_ref, vi_ref):
    xb = x_ref[0]
    acc = jnp.dot(w_ref[...], xb, preferred_element_type=jnp.float32)
    acc = acc + b_ref[...]
    mpib = mpi_ref[0]
    nt = nt_ref[...]
    vi = jnp.broadcast_to(nt[0:1], (OUT_F, CH))
    for j in range(1, NEIGH):
        vi = jnp.where(mpib == j, nt[j:j + 1], vi)
    col = pl.program_id(1) * CH + lax.broadcasted_iota(jnp.int32, (OUT_F, CH), 1)
    vi = jnp.where(col < N_RAW, vi, -1)
    xfc_ref[0] = acc
    vi_ref[0] = vi


def _fc_vi(x, w, b2, mpi, neigh_t):
    grid = (BATCH, NCHUNK)
    return pl.pallas_call(
        _fc_vi_kernel,
        grid=grid,
        in_specs=[
            pl.BlockSpec((1, IN_F, CH), lambda s, c: (s, 0, c)),
            pl.BlockSpec((OUT_F, IN_F), lambda s, c: (0, 0)),
            pl.BlockSpec((OUT_F, 1), lambda s, c: (0, 0)),
            pl.BlockSpec((1, OUT_F, CH), lambda s, c: (s, 0, c)),
            pl.BlockSpec((8, CH), lambda s, c: (0, c)),
        ],
        out_specs=[
            pl.BlockSpec((1, OUT_F, CH), lambda s, c: (s, 0, c)),
            pl.BlockSpec((1, OUT_F, CH), lambda s, c: (s, 0, c)),
        ],
        out_shape=[
            jax.ShapeDtypeStruct((BATCH, OUT_F, W_P), jnp.float32),
            jax.ShapeDtypeStruct((BATCH, OUT_F, W_P), jnp.int32),
        ],
    )(x, w, b2, mpi, neigh_t)


def _scatter_kernel(vi_hbm, xf_hbm, y_hbm, vbuf, vich, xfch, s0, s1, s2, s3, so):
    wid = lax.axis_index("s") * NC + lax.axis_index("c")
    sems = ((s0, s1), (s2, s3))
    zero16 = jnp.zeros((16,), jnp.float32)

    def issue(sample, f, ci, slot):
        pltpu.async_copy(vi_hbm.at[sample, f, pl.ds(ci * CH, CH)],
                         vich.at[slot], sems[0][slot])
        pltpu.async_copy(xf_hbm.at[sample, f, pl.ds(ci * CH, CH)],
                         xfch.at[slot], sems[1][slot])

    def wait(slot):
        pltpu.make_async_copy(vi_hbm.at[0, 0, pl.ds(0, CH)],
                              vich.at[slot], sems[0][slot]).wait()
        pltpu.make_async_copy(xf_hbm.at[0, 0, pl.ds(0, CH)],
                              xfch.at[slot], sems[1][slot]).wait()

    for t in range(TASKS):
        half = t & 1
        f = wid * (TASKS // 2) + (t >> 1)
        lo = HALF0 * half
        ln = HALF1 if half else HALF0

        def zfill(i, carry):
            base = i * 64
            vbuf[pl.ds(base, 16)] = zero16
            vbuf[pl.ds(base + 16, 16)] = zero16
            vbuf[pl.ds(base + 32, 16)] = zero16
            vbuf[pl.ds(base + 48, 16)] = zero16
            return carry
        lax.fori_loop(0, VBUF // 64, zfill, 0)

        def process(slot, ci):
            def body(k, carry):
                o = k * 64
                for u in range(4):
                    viv = vich[slot, pl.ds(o + u * 16, 16)]
                    xfv = xfch[slot, pl.ds(o + u * 16, 16)]
                    rel = viv - lo
                    m = plsc.bitcast(rel, jnp.uint32) < jnp.uint32(ln)
                    plsc.store_scatter(vbuf, [rel], xfv, mask=m)
                return carry
            lax.fori_loop(0, CH // 64, body, 0)

        for s in range(BATCH):
            issue(s, f, 0, 0)

            def outer(ci2, carry, s=s, f=f):
                for bslot in range(2):
                    ci = ci2 + bslot
                    issue(s, f, ci + 1, 1 - bslot)
                    wait(bslot)
                    process(bslot, ci)
                return carry
            lax.fori_loop(0, (NCHUNK - 1) // 2,
                          lambda i, c, fn=outer: fn(i * 2, c), 0)

            wait(0)
            process(0, NCHUNK - 1)

        for s in range(BATCH):
            pltpu.async_copy(vbuf.at[pl.ds(0, ln)],
                             y_hbm.at[s, f, pl.ds(lo, ln)], so)
        for s in range(BATCH):
            pltpu.make_async_copy(vbuf.at[pl.ds(0, ln)],
                                  y_hbm.at[s, f, pl.ds(lo, ln)], so).wait()


def _scatter(vi2, xf2):
    k = functools.partial(
        pl.kernel,
        mesh=plsc.VectorSubcoreMesh(core_axis_name="c", subcore_axis_name="s"),
        out_type=jax.ShapeDtypeStruct((BATCH, OUT_F, N_VERT), jnp.float32),
        scratch_types=[
            pltpu.VMEM((VBUF,), jnp.float32),
            pltpu.VMEM((2, CH), jnp.int32),
            pltpu.VMEM((2, CH), jnp.float32),
            pltpu.SemaphoreType.DMA,
            pltpu.SemaphoreType.DMA,
            pltpu.SemaphoreType.DMA,
            pltpu.SemaphoreType.DMA,
            pltpu.SemaphoreType.DMA,
        ],
        compiler_params=pltpu.CompilerParams(use_tc_tiling_on_sc=False,
                                             needs_layout_passes=False),
    )(_scatter_kernel)
    return k(vi2, xf2)


@jax.jit
def kernel(x, max_pool_indices, up_neigh_indices, down_indices, W, b):
    up_flat = jnp.reshape(up_neigh_indices.astype(jnp.int32), (-1,))
    down_p = jnp.concatenate(
        [down_indices.astype(jnp.int32),
         jnp.zeros((W_P - N_RAW,), jnp.int32)])
    neigh_t = _neigh_gather(up_flat, down_p)

    xfc_p, vi_p = _fc_vi(x, W, b.reshape(OUT_F, 1),
                         max_pool_indices.astype(jnp.int32), neigh_t)

    return _scatter(vi_p, xfc_p)
